# Initial kernel scaffold; baseline (speedup 1.0000x reference)
import functools

import jax
import jax.numpy as jnp
from jax import lax
from jax.experimental import pallas as pl
from jax.experimental.pallas import tpu as pltpu
from jax.experimental.pallas import tpu_sc as plsc

N = 10000
D = 128
E = 320000
EMB = 10
EMBP = 16

NC = 2
NS = 16
NW = NC * NS
LANES = 16

EDGES_PER_TILE = E // NW
BATCH = 80
NBATCH = EDGES_PER_TILE // BATCH

ROW_BLK = 400
GRID = N // ROW_BLK

WB_ROWS = N // NS
WB_CHUNK = 125


def _prep_body(alpha_ref, x_ref, w0_ref, w1_ref, w2_ref, e2_ref,
               f0_ref, f1_ref, f2_ref, m_ref, acc_ref):
    i = pl.program_id(0)
    x = x_ref[...]
    dot = functools.partial(jnp.dot, preferred_element_type=jnp.float32,
                            precision=lax.Precision.HIGHEST)
    f0_ref[...] = dot(x, w0_ref[...])
    f1_ref[...] = dot(x, w1_ref[...])
    f2_ref[...] = dot(x, w2_ref[...])
    contrib = lax.dot_general(e2_ref[...], x, (((0,), (0,)), ((), ())),
                              preferred_element_type=jnp.float32,
                              precision=lax.Precision.HIGHEST)

    @pl.when(i == 0)
    def _():
        acc_ref[...] = contrib

    @pl.when(i != 0)
    def _():
        acc_ref[...] = acc_ref[...] + contrib

    @pl.when(i == GRID - 1)
    def _():
        wsum = w0_ref[...] + w1_ref[...] + w2_ref[...]
        m_ref[...] = alpha_ref[0] * dot(acc_ref[...], wsum)


def _dense_prep(x, w0, w1, w2, e2p, alpha):
    alpha1 = jnp.reshape(alpha, (1,))
    return pl.pallas_call(
        _prep_body,
        grid=(GRID,),
        in_specs=[
            pl.BlockSpec(memory_space=pltpu.SMEM),
            pl.BlockSpec((ROW_BLK, D), lambda i: (i, 0)),
            pl.BlockSpec((D, D), lambda i: (0, 0)),
            pl.BlockSpec((D, D), lambda i: (0, 0)),
            pl.BlockSpec((D, D), lambda i: (0, 0)),
            pl.BlockSpec((ROW_BLK, EMBP), lambda i: (i, 0)),
        ],
        out_specs=[
            pl.BlockSpec((ROW_BLK, D), lambda i: (i, 0)),
            pl.BlockSpec((ROW_BLK, D), lambda i: (i, 0)),
            pl.BlockSpec((ROW_BLK, D), lambda i: (i, 0)),
            pl.BlockSpec((EMBP, D), lambda i: (0, 0)),
        ],
        out_shape=[
            jax.ShapeDtypeStruct((N, D), jnp.float32),
            jax.ShapeDtypeStruct((N, D), jnp.float32),
            jax.ShapeDtypeStruct((N, D), jnp.float32),
            jax.ShapeDtypeStruct((EMBP, D), jnp.float32),
        ],
        scratch_shapes=[pltpu.VMEM((EMBP, D), jnp.float32)],
    )(alpha1, x, w0, w1, w2, e2p)


_MESH = plsc.VectorSubcoreMesh(core_axis_name="c", subcore_axis_name="s")


@functools.partial(
    pl.kernel,
    out_type=jax.ShapeDtypeStruct((NC, N, D), jnp.float32),
    mesh=_MESH,
    scratch_types=[
        pltpu.VMEM_SHARED((N, D), jnp.float32),
        pltpu.VMEM((BATCH,), jnp.int32),
        pltpu.VMEM((BATCH,), jnp.int32),
        pltpu.VMEM((BATCH,), jnp.float32),
        pltpu.VMEM((BATCH, D), jnp.float32),
        pltpu.VMEM((WB_CHUNK, D), jnp.float32),
    ],
)
def _sc_segment(f0, f1, f2, ei0, ei1, ei2, ew0, ew1, ew2, out_hbm,
                acc_sh, src_v, dst_v, w_v, rows_v, zbuf):
    c = lax.axis_index("c")
    s = lax.axis_index("s")
    wid = s * NC + c

    zvec = jnp.zeros((LANES,), jnp.float32)

    @pl.loop(0, WB_CHUNK)
    def _(r):
        for cc in range(D // LANES):
            zbuf[r, pl.ds(cc * LANES, LANES)] = zvec

    row0 = s * WB_ROWS
    for r in range(WB_ROWS // WB_CHUNK):
        pltpu.sync_copy(zbuf, acc_sh.at[pl.ds(row0 + r * WB_CHUNK, WB_CHUNK)])
    plsc.subcore_barrier()

    base0 = wid * EDGES_PER_TILE
    for f, ei, ew in ((f0, ei0, ew0), (f1, ei1, ew1), (f2, ei2, ew2)):
        @pl.loop(0, NBATCH)
        def _(j):
            base = base0 + j * BATCH
            pltpu.sync_copy(ei.at[0, pl.ds(base, BATCH)], src_v)
            pltpu.sync_copy(ei.at[1, pl.ds(base, BATCH)], dst_v)
            pltpu.sync_copy(ew.at[pl.ds(base, BATCH)], w_v)
            pltpu.sync_copy(f.at[src_v], rows_v)

            @pl.loop(0, BATCH)
            def _(i):
                wvec = jnp.full((LANES,), w_v[i], jnp.float32)
                for cc in range(D // LANES):
                    sl = (i, pl.ds(cc * LANES, LANES))
                    rows_v[sl] = rows_v[sl] * wvec

            pltpu.sync_copy(rows_v, acc_sh.at[dst_v], add=True)

    plsc.subcore_barrier()
    for r in range(WB_ROWS // WB_CHUNK):
        sl = pl.ds(row0 + r * WB_CHUNK, WB_CHUNK)
        pltpu.sync_copy(acc_sh.at[sl], out_hbm.at[c, sl])


def _final_body(acc_ref, e1_ref, m_ref, o_ref):
    learned = jnp.dot(e1_ref[...], m_ref[...],
                      preferred_element_type=jnp.float32,
                      precision=lax.Precision.HIGHEST)
    o_ref[...] = jnp.maximum(acc_ref[0] + acc_ref[1] + learned, 0.0)


def _finalize(acc, e1p, m):
    return pl.pallas_call(
        _final_body,
        grid=(GRID,),
        in_specs=[
            pl.BlockSpec((NC, ROW_BLK, D), lambda i: (0, i, 0)),
            pl.BlockSpec((ROW_BLK, EMBP), lambda i: (i, 0)),
            pl.BlockSpec((EMBP, D), lambda i: (0, 0)),
        ],
        out_specs=pl.BlockSpec((ROW_BLK, D), lambda i: (i, 0)),
        out_shape=jax.ShapeDtypeStruct((N, D), jnp.float32),
    )(acc, e1p, m)


def kernel(node_features, edge_index_0, edge_weight_0, edge_index_1,
           edge_weight_1, edge_index_2, edge_weight_2, W0, W1, W2,
           embed1, embed2, alpha):
    e1p = jnp.pad(embed1, ((0, 0), (0, EMBP - EMB)))
    e2p = jnp.pad(embed2, ((0, 0), (0, EMBP - EMB)))
    f0, f1, f2, m = _dense_prep(node_features, W0, W1, W2, e2p, alpha)
    acc = _sc_segment(f0, f1, f2, edge_index_0, edge_index_1, edge_index_2,
                      edge_weight_0, edge_weight_1, edge_weight_2)
    return _finalize(acc, e1p, m)


# speedup vs baseline: 3.5976x; 3.5976x over previous
import functools

import jax
import jax.numpy as jnp
from jax import lax
from jax.experimental import pallas as pl
from jax.experimental.pallas import tpu as pltpu
from jax.experimental.pallas import tpu_sc as plsc

N = 10000
D = 128
E = 320000
EMB = 10
EMBP = 16

NC = 2
NS = 16
NW = NC * NS
LANES = 16

EDGES_PER_TILE = E // NW
BATCH = 80
NBATCH = EDGES_PER_TILE // BATCH

ROW_BLK = 400
GRID = N // ROW_BLK

WB_CHUNK = 200
WB_NCHUNK = N // WB_CHUNK


def _prep_body(alpha_ref, x_ref, w0_ref, w1_ref, w2_ref, e2_ref,
               f0_ref, f1_ref, f2_ref, m_ref, acc_ref):
    i = pl.program_id(0)
    x = x_ref[...]
    dot = functools.partial(jnp.dot, preferred_element_type=jnp.float32,
                            precision=lax.Precision.HIGHEST)
    f0_ref[...] = dot(x, w0_ref[...])
    f1_ref[...] = dot(x, w1_ref[...])
    f2_ref[...] = dot(x, w2_ref[...])
    contrib = lax.dot_general(e2_ref[...], x, (((0,), (0,)), ((), ())),
                              preferred_element_type=jnp.float32,
                              precision=lax.Precision.HIGHEST)

    @pl.when(i == 0)
    def _():
        acc_ref[...] = contrib

    @pl.when(i != 0)
    def _():
        acc_ref[...] = acc_ref[...] + contrib

    @pl.when(i == GRID - 1)
    def _():
        wsum = w0_ref[...] + w1_ref[...] + w2_ref[...]
        m_ref[...] = alpha_ref[0] * dot(acc_ref[...], wsum)


def _dense_prep(x, w0, w1, w2, e2p, alpha):
    alpha1 = jnp.reshape(alpha, (1,))
    return pl.pallas_call(
        _prep_body,
        grid=(GRID,),
        in_specs=[
            pl.BlockSpec(memory_space=pltpu.SMEM),
            pl.BlockSpec((ROW_BLK, D), lambda i: (i, 0)),
            pl.BlockSpec((D, D), lambda i: (0, 0)),
            pl.BlockSpec((D, D), lambda i: (0, 0)),
            pl.BlockSpec((D, D), lambda i: (0, 0)),
            pl.BlockSpec((ROW_BLK, EMBP), lambda i: (i, 0)),
        ],
        out_specs=[
            pl.BlockSpec((ROW_BLK, D), lambda i: (i, 0)),
            pl.BlockSpec((ROW_BLK, D), lambda i: (i, 0)),
            pl.BlockSpec((ROW_BLK, D), lambda i: (i, 0)),
            pl.BlockSpec((EMBP, D), lambda i: (0, 0)),
        ],
        out_shape=[
            jax.ShapeDtypeStruct((N, D), jnp.float32),
            jax.ShapeDtypeStruct((N, D), jnp.float32),
            jax.ShapeDtypeStruct((N, D), jnp.float32),
            jax.ShapeDtypeStruct((EMBP, D), jnp.float32),
        ],
        scratch_shapes=[pltpu.VMEM((EMBP, D), jnp.float32)],
    )(alpha1, x, w0, w1, w2, e2p)


_MESH = plsc.VectorSubcoreMesh(core_axis_name="c", subcore_axis_name="s")


@functools.partial(
    pl.kernel,
    out_type=jax.ShapeDtypeStruct((NC, N, D), jnp.float32),
    mesh=_MESH,
    scratch_types=[
        pltpu.VMEM_SHARED((N, D), jnp.float32),
        pltpu.VMEM((BATCH,), jnp.int32),
        pltpu.VMEM((BATCH,), jnp.int32),
        pltpu.VMEM((BATCH,), jnp.float32),
        pltpu.VMEM((BATCH, D), jnp.float32),
        pltpu.VMEM((WB_CHUNK, D), jnp.float32),
    ],
)
def _sc_segment(f0, f1, f2, es0, ed0, es1, ed1, es2, ed2, ew0, ew1, ew2,
                out_hbm, acc_sh, src_v, dst_v, w_v, rows_v, zbuf):
    c = lax.axis_index("c")
    s = lax.axis_index("s")
    wid = s * NC + c

    zvec = jnp.zeros((LANES,), jnp.float32)

    @pl.loop(0, WB_CHUNK)
    def _(r):
        for cc in range(D // LANES):
            zbuf[r, pl.ds(cc * LANES, LANES)] = zvec

    for k in range((WB_NCHUNK + NS - 1) // NS):
        cid = s + NS * k

        @pl.when(cid < WB_NCHUNK)
        def _():
            pltpu.sync_copy(zbuf, acc_sh.at[pl.ds(cid * WB_CHUNK, WB_CHUNK)])
    plsc.subcore_barrier()

    base0 = wid * EDGES_PER_TILE
    for f, es, ed, ew in ((f0, es0, ed0, ew0), (f1, es1, ed1, ew1),
                          (f2, es2, ed2, ew2)):
        @pl.loop(0, NBATCH)
        def _(j):
            base = base0 + j * BATCH
            pltpu.sync_copy(es.at[pl.ds(base, BATCH)], src_v)
            pltpu.sync_copy(ed.at[pl.ds(base, BATCH)], dst_v)
            pltpu.sync_copy(ew.at[pl.ds(base, BATCH)], w_v)
            pltpu.sync_copy(f.at[src_v], rows_v)

            @pl.loop(0, BATCH // LANES)
            def _(g):
                wgrp = w_v[pl.ds(g * LANES, LANES)]
                for r in range(LANES):
                    wvec = jnp.full((LANES,), wgrp[r], jnp.float32)
                    row = g * LANES + r
                    for cc in range(D // LANES):
                        sl = (row, pl.ds(cc * LANES, LANES))
                        rows_v[sl] = rows_v[sl] * wvec

            pltpu.sync_copy(rows_v, acc_sh.at[dst_v], add=True)

    plsc.subcore_barrier()
    for k in range((WB_NCHUNK + NS - 1) // NS):
        cid = s + NS * k

        @pl.when(cid < WB_NCHUNK)
        def _():
            sl = pl.ds(cid * WB_CHUNK, WB_CHUNK)
            pltpu.sync_copy(acc_sh.at[sl], out_hbm.at[c, sl])


def _final_body(acc_ref, e1_ref, m_ref, o_ref):
    learned = jnp.dot(e1_ref[...], m_ref[...],
                      preferred_element_type=jnp.float32,
                      precision=lax.Precision.HIGHEST)
    o_ref[...] = jnp.maximum(acc_ref[0] + acc_ref[1] + learned, 0.0)


def _finalize(acc, e1p, m):
    return pl.pallas_call(
        _final_body,
        grid=(GRID,),
        in_specs=[
            pl.BlockSpec((NC, ROW_BLK, D), lambda i: (0, i, 0)),
            pl.BlockSpec((ROW_BLK, EMBP), lambda i: (i, 0)),
            pl.BlockSpec((EMBP, D), lambda i: (0, 0)),
        ],
        out_specs=pl.BlockSpec((ROW_BLK, D), lambda i: (i, 0)),
        out_shape=jax.ShapeDtypeStruct((N, D), jnp.float32),
    )(acc, e1p, m)


def kernel(node_features, edge_index_0, edge_weight_0, edge_index_1,
           edge_weight_1, edge_index_2, edge_weight_2, W0, W1, W2,
           embed1, embed2, alpha):
    e1p = jnp.pad(embed1, ((0, 0), (0, EMBP - EMB)))
    e2p = jnp.pad(embed2, ((0, 0), (0, EMBP - EMB)))
    f0, f1, f2, m = _dense_prep(node_features, W0, W1, W2, e2p, alpha)
    acc = _sc_segment(f0, f1, f2,
                      edge_index_0[0], edge_index_0[1],
                      edge_index_1[0], edge_index_1[1],
                      edge_index_2[0], edge_index_2[1],
                      edge_weight_0, edge_weight_1, edge_weight_2)
    return _finalize(acc, e1p, m)
